# baseline (speedup 1.0000x reference)
import jax
import jax.numpy as jnp
from jax import lax
from jax.experimental import pallas as pl
from jax.experimental.pallas import tpu as pltpu
from jax.experimental.pallas import tpu_sc as plsc

N_NODES = 50000
N_CELLS = 100000
D = 128

NUM_CORES = 2
NUM_SUBCORES = 16
NUM_TILES = NUM_CORES * NUM_SUBCORES

CHUNK_CELLS = 1024
HALF_CELLS = CHUNK_CELLS // 2
NUM_CHUNKS = -(-N_CELLS // CHUNK_CELLS)
LAST_CHUNK = NUM_CHUNKS - 1
LAST_CELLS = N_CELLS - LAST_CHUNK * CHUNK_CELLS
GROUPS = CHUNK_CELLS // 16
SCATTER_BYTES = 3 * CHUNK_CELLS * 4


def _q_matmul_body(u_ref, w_ref, q_ref):
    u = u_ref[...]
    qs = [jnp.dot(u, w_ref[9 + j * D:9 + (j + 1) * D, :],
                  preferred_element_type=jnp.float32) for j in range(3)]
    blk = u.shape[0]
    q_ref[...] = jnp.concatenate(
        qs + [jnp.zeros((blk, 7), jnp.float32)], axis=1)


def _compute_q(u2d, w):
    blk = 10000
    return pl.pallas_call(
        _q_matmul_body,
        grid=(N_NODES // blk,),
        in_specs=[
            pl.BlockSpec((blk, D), lambda i: (i, 0)),
            pl.BlockSpec((393, 3), lambda i: (0, 0)),
        ],
        out_specs=pl.BlockSpec((blk, 16), lambda i: (i, 0)),
        out_shape=jax.ShapeDtypeStruct((N_NODES, 16), jnp.float32),
    )(u2d, w)


def _sc_body(q_hbm, tri_hbm, cc_hbm, vp0_hbm, vp1_hbm, vp2_hbm, ffd_hbm,
             wtab_hbm, zeros_hbm, out0_hbm, out1_hbm,
             idx0_v, idx1_v, rows_v, cc_v, vp_v, ffd_v, con0_v, con1_v,
             wtab_v, sem, ssem, dsem, accum_sh):
    core = lax.axis_index("c")
    sub = lax.axis_index("s")
    wid = core * NUM_SUBCORES + sub

    pltpu.sync_copy(wtab_hbm, wtab_v)

    @pl.when(sub == 0)
    def _():
        pltpu.sync_copy(zeros_hbm, accum_sh)

    plsc.subcore_barrier()

    wv = [wtab_v[r] for r in range(27)]

    iota = lax.iota(jnp.int32, 16)
    lane_off = [jnp.full((16,), 3 * j + i, jnp.int32)
                for j in range(3) for i in range(3)]
    zero16 = jnp.zeros((16,), jnp.float32)

    def drain_prev_scatter():
        pltpu.make_async_copy(q_hbm.at[pl.ds(0, SCATTER_BYTES // 64)],
                              rows_v.at[pl.ds(0, SCATTER_BYTES // 64)],
                              ssem).wait()

    def do_chunk(ch, idx_v, contrib_v, first):
        c0 = ch * CHUNK_CELLS

        def stage(ncells):
            ds = []
            for src, dst in ((tri_hbm, idx_v), (cc_hbm, cc_v),
                             (vp0_hbm, vp_v.at[pl.ds(0, 2)]),
                             (vp1_hbm, vp_v.at[pl.ds(2, 2)]),
                             (vp2_hbm, vp_v.at[pl.ds(4, 2)]),
                             (ffd_hbm, ffd_v)):
                ds.append(pltpu.async_copy(
                    src.at[:, pl.ds(c0, ncells)],
                    dst.at[:, pl.ds(0, ncells)], dsem))
            for d in ds:
                d.wait()

        @pl.when(ch != LAST_CHUNK)
        def _():
            stage(CHUNK_CELLS)

        @pl.when(ch == LAST_CHUNK)
        def _():
            stage(LAST_CELLS)

        def fire_gathers(h):
            ds = []
            for j in range(3):
                ds.append(pltpu.async_copy(
                    q_hbm.at[idx_v.at[j, pl.ds(h * HALF_CELLS, HALF_CELLS)]],
                    rows_v.at[pl.ds(j * CHUNK_CELLS + h * HALF_CELLS,
                                    HALF_CELLS)],
                    sem))
            return ds

        g0 = fire_gathers(0)
        g1 = fire_gathers(1)

        @pl.when(jnp.logical_not(first))
        def _():
            drain_prev_scatter()

        def compute_half(h):
            for g in range(h * (GROUPS // 2), (h + 1) * (GROUPS // 2)):
                o = g * 16
                rowj = [iota + (j * CHUNK_CELLS + o) for j in range(3)]
                cc0 = cc_v[0, pl.ds(o, 16)]
                cc1 = cc_v[1, pl.ds(o, 16)]
                vps = [vp_v[k, pl.ds(o, 16)] for k in range(6)]
                for i in range(3):
                    s0 = plsc.load_gather(rows_v, [rowj[0], lane_off[0 + i]])
                    s1 = plsc.load_gather(rows_v, [rowj[1], lane_off[3 + i]])
                    s2 = plsc.load_gather(rows_v, [rowj[2], lane_off[6 + i]])
                    base = wv[24 + i] + cc0 * wv[0 + i] + cc1 * wv[3 + i]
                    for k in range(6):
                        base = base + vps[k] * wv[6 + 3 * k + i]
                    ffd_i = ffd_v[i, pl.ds(o, 16)]
                    contrib_v[i, pl.ds(o, 16)] = \
                        ffd_i * (base + (s0 + s1) + s2)

        def fire_scatters(h):
            for i in range(3):
                pltpu.async_copy(
                    contrib_v.at[i, pl.ds(h * HALF_CELLS, HALF_CELLS)],
                    accum_sh.at[idx_v.at[i, pl.ds(h * HALF_CELLS,
                                                  HALF_CELLS)]],
                    ssem, add=True)

        for d in g0:
            d.wait()
        compute_half(0)
        fire_scatters(0)

        for d in g1:
            d.wait()
        compute_half(1)

        @pl.when(ch == LAST_CHUNK)
        def _():
            for i in range(3):
                for tt in range(LAST_CELLS // 16, GROUPS):
                    contrib_v[i, pl.ds(tt * 16, 16)] = zero16

        fire_scatters(1)

    def chunk_body(m, _):
        ch0 = wid + NUM_TILES * (2 * m)
        ch1 = jnp.where(2 * m + 1 == 3,
                        jnp.where(wid == 0, 96,
                                  jnp.where(wid == 16, 97, NUM_CHUNKS)),
                        wid + NUM_TILES * (2 * m + 1))

        @pl.when(ch0 < NUM_CHUNKS)
        def _():
            do_chunk(ch0, idx0_v, con0_v, m == 0)

        @pl.when(ch1 < NUM_CHUNKS)
        def _():
            do_chunk(ch1, idx1_v, con1_v, False)

        return ()

    lax.fori_loop(0, 2, chunk_body, ())

    drain_prev_scatter()

    plsc.subcore_barrier()

    @pl.when(jnp.logical_and(sub == 0, core == 0))
    def _():
        pltpu.sync_copy(accum_sh, out0_hbm)

    @pl.when(jnp.logical_and(sub == 0, core == 1))
    def _():
        pltpu.sync_copy(accum_sh, out1_hbm)


def _sc_scatter(q, tri_t, cc_t, vp0, vp1, vp2, ffd_t, wtab, zeros):
    mesh = plsc.VectorSubcoreMesh(core_axis_name="c", subcore_axis_name="s")
    kern = pl.kernel(
        _sc_body,
        out_type=(jax.ShapeDtypeStruct((N_NODES,), jnp.float32),
                  jax.ShapeDtypeStruct((N_NODES,), jnp.float32)),
        mesh=mesh,
        compiler_params=pltpu.CompilerParams(needs_layout_passes=False,
                                             use_tc_tiling_on_sc=False),
        scratch_types=[
            pltpu.VMEM((3, CHUNK_CELLS), jnp.int32),
            pltpu.VMEM((3, CHUNK_CELLS), jnp.int32),
            pltpu.VMEM((3 * CHUNK_CELLS, 16), jnp.float32),
            pltpu.VMEM((2, CHUNK_CELLS), jnp.float32),
            pltpu.VMEM((6, CHUNK_CELLS), jnp.float32),
            pltpu.VMEM((3, CHUNK_CELLS), jnp.float32),
            pltpu.VMEM((3, CHUNK_CELLS), jnp.float32),
            pltpu.VMEM((3, CHUNK_CELLS), jnp.float32),
            pltpu.VMEM((32, 16), jnp.float32),
            pltpu.SemaphoreType.DMA,
            pltpu.SemaphoreType.DMA,
            pltpu.SemaphoreType.DMA,
            pltpu.VMEM_SHARED((N_NODES,), jnp.float32),
        ],
    )
    return kern(q, tri_t, cc_t, vp0, vp1, vp2, ffd_t, wtab, zeros)


def _combine_body(p0_ref, p1_ref, m_ref, o_ref):
    o_ref[...] = (p0_ref[...] + p1_ref[...]) * m_ref[...]


def _combine(p0, p1, inv_mass2d):
    return pl.pallas_call(
        _combine_body,
        out_shape=jax.ShapeDtypeStruct((1, N_NODES), jnp.float32),
    )(p0[None, :], p1[None, :], inv_mass2d)


@jax.jit
def kernel(u, t, triangulation, cell_centers, cell_local_vertex_pos,
           free_form_data, inv_mass, W, b):
    u2d = u[0]

    q = _compute_q(u2d, W)

    const = t[0, 0] * W[0] + b
    wrows = jnp.concatenate([W[1:9].reshape(-1), const,
                             jnp.zeros((5,), jnp.float32)])
    wtab = jnp.broadcast_to(wrows[:, None], (32, 16))

    zeros = jnp.zeros((N_NODES,), jnp.float32)

    tri_t = triangulation.T
    cc_t = cell_centers.T
    vp0 = cell_local_vertex_pos[:, 0, :].T
    vp1 = cell_local_vertex_pos[:, 1, :].T
    vp2 = cell_local_vertex_pos[:, 2, :].T
    ffd_t = free_form_data.T

    p0, p1 = _sc_scatter(q, tri_t, cc_t, vp0, vp1, vp2, ffd_t, wtab, zeros)

    return _combine(p0, p1, inv_mass[None, :])


# speedup vs baseline: 1.0551x; 1.0551x over previous
import jax
import jax.numpy as jnp
from jax import lax
from jax.experimental import pallas as pl
from jax.experimental.pallas import tpu as pltpu
from jax.experimental.pallas import tpu_sc as plsc

N_NODES = 50000
N_CELLS = 100000
D = 128

NUM_CORES = 2
NUM_SUBCORES = 16
NUM_TILES = NUM_CORES * NUM_SUBCORES

CHUNK_CELLS = 1024
HALF_CELLS = CHUNK_CELLS // 2
NUM_CHUNKS = -(-N_CELLS // CHUNK_CELLS)
LAST_CHUNK = NUM_CHUNKS - 1
LAST_CELLS = N_CELLS - LAST_CHUNK * CHUNK_CELLS
GROUPS = CHUNK_CELLS // 16
SCATTER_BYTES = 3 * CHUNK_CELLS * 4


def _q_matmul_body(u_ref, w_ref, q_ref):
    q_ref[...] = jnp.dot(u_ref[...], w_ref[...],
                         preferred_element_type=jnp.float32)


def _compute_q(u2d, wcat):
    blk = 10000
    return pl.pallas_call(
        _q_matmul_body,
        grid=(N_NODES // blk,),
        in_specs=[
            pl.BlockSpec((blk, D), lambda i: (i, 0)),
            pl.BlockSpec((D, 16), lambda i: (0, 0)),
        ],
        out_specs=pl.BlockSpec((blk, 16), lambda i: (i, 0)),
        out_shape=jax.ShapeDtypeStruct((N_NODES, 16), jnp.float32),
    )(u2d, wcat)


def _sc_body(q_hbm, tri_hbm, cc_hbm, vp0_hbm, vp1_hbm, vp2_hbm, ffd_hbm,
             wtab_hbm, zeros_hbm, out0_hbm, out1_hbm,
             idx0_v, idx1_v, rows_v, cc_v, vp_v, ffd_v, con0_v, con1_v,
             wtab_v, sem, ssem, dsem, accum_sh):
    core = lax.axis_index("c")
    sub = lax.axis_index("s")
    wid = core * NUM_SUBCORES + sub

    pltpu.sync_copy(wtab_hbm, wtab_v)

    @pl.when(sub == 0)
    def _():
        pltpu.sync_copy(zeros_hbm, accum_sh)

    plsc.subcore_barrier()

    wv = [wtab_v[r] for r in range(27)]

    iota = lax.iota(jnp.int32, 16)
    lane_off = [jnp.full((16,), 3 * j + i, jnp.int32)
                for j in range(3) for i in range(3)]
    zero16 = jnp.zeros((16,), jnp.float32)

    def drain_prev_scatter():
        pltpu.make_async_copy(q_hbm.at[pl.ds(0, SCATTER_BYTES // 64)],
                              rows_v.at[pl.ds(0, SCATTER_BYTES // 64)],
                              ssem).wait()

    def do_chunk(ch, idx_v, contrib_v, first):
        c0 = ch * CHUNK_CELLS

        def stage(ncells):
            ds = []
            for src, dst in ((tri_hbm, idx_v), (cc_hbm, cc_v),
                             (vp0_hbm, vp_v.at[pl.ds(0, 2)]),
                             (vp1_hbm, vp_v.at[pl.ds(2, 2)]),
                             (vp2_hbm, vp_v.at[pl.ds(4, 2)]),
                             (ffd_hbm, ffd_v)):
                ds.append(pltpu.async_copy(
                    src.at[:, pl.ds(c0, ncells)],
                    dst.at[:, pl.ds(0, ncells)], dsem))
            for d in ds:
                d.wait()

        @pl.when(ch != LAST_CHUNK)
        def _():
            stage(CHUNK_CELLS)

        @pl.when(ch == LAST_CHUNK)
        def _():
            stage(LAST_CELLS)

        def fire_gathers(h):
            ds = []
            for j in range(3):
                ds.append(pltpu.async_copy(
                    q_hbm.at[idx_v.at[j, pl.ds(h * HALF_CELLS, HALF_CELLS)]],
                    rows_v.at[pl.ds(j * CHUNK_CELLS + h * HALF_CELLS,
                                    HALF_CELLS)],
                    sem))
            return ds

        g0 = fire_gathers(0)
        g1 = fire_gathers(1)

        @pl.when(jnp.logical_not(first))
        def _():
            drain_prev_scatter()

        def compute_half(h):
            for g in range(h * (GROUPS // 2), (h + 1) * (GROUPS // 2)):
                o = g * 16
                rowj = [iota + (j * CHUNK_CELLS + o) for j in range(3)]
                cc0 = cc_v[0, pl.ds(o, 16)]
                cc1 = cc_v[1, pl.ds(o, 16)]
                vps = [vp_v[k, pl.ds(o, 16)] for k in range(6)]
                for i in range(3):
                    s0 = plsc.load_gather(rows_v, [rowj[0], lane_off[0 + i]])
                    s1 = plsc.load_gather(rows_v, [rowj[1], lane_off[3 + i]])
                    s2 = plsc.load_gather(rows_v, [rowj[2], lane_off[6 + i]])
                    base = wv[24 + i] + cc0 * wv[0 + i] + cc1 * wv[3 + i]
                    for k in range(6):
                        base = base + vps[k] * wv[6 + 3 * k + i]
                    ffd_i = ffd_v[i, pl.ds(o, 16)]
                    contrib_v[i, pl.ds(o, 16)] = \
                        ffd_i * (base + (s0 + s1) + s2)

        def fire_scatters(h):
            for i in range(3):
                pltpu.async_copy(
                    contrib_v.at[i, pl.ds(h * HALF_CELLS, HALF_CELLS)],
                    accum_sh.at[idx_v.at[i, pl.ds(h * HALF_CELLS,
                                                  HALF_CELLS)]],
                    ssem, add=True)

        for d in g0:
            d.wait()
        compute_half(0)
        fire_scatters(0)

        for d in g1:
            d.wait()
        compute_half(1)

        @pl.when(ch == LAST_CHUNK)
        def _():
            for i in range(3):
                for tt in range(LAST_CELLS // 16, GROUPS):
                    contrib_v[i, pl.ds(tt * 16, 16)] = zero16

        fire_scatters(1)

    def chunk_body(m, _):
        ch0 = wid + NUM_TILES * (2 * m)
        ch1 = jnp.where(2 * m + 1 == 3,
                        jnp.where(wid == 0, 96,
                                  jnp.where(wid == 16, 97, NUM_CHUNKS)),
                        wid + NUM_TILES * (2 * m + 1))

        @pl.when(ch0 < NUM_CHUNKS)
        def _():
            do_chunk(ch0, idx0_v, con0_v, m == 0)

        @pl.when(ch1 < NUM_CHUNKS)
        def _():
            do_chunk(ch1, idx1_v, con1_v, False)

        return ()

    lax.fori_loop(0, 2, chunk_body, ())

    drain_prev_scatter()

    plsc.subcore_barrier()

    @pl.when(jnp.logical_and(sub == 0, core == 0))
    def _():
        pltpu.sync_copy(accum_sh, out0_hbm)

    @pl.when(jnp.logical_and(sub == 0, core == 1))
    def _():
        pltpu.sync_copy(accum_sh, out1_hbm)


def _sc_scatter(q, tri_t, cc_t, vp0, vp1, vp2, ffd_t, wtab, zeros):
    mesh = plsc.VectorSubcoreMesh(core_axis_name="c", subcore_axis_name="s")
    kern = pl.kernel(
        _sc_body,
        out_type=(jax.ShapeDtypeStruct((N_NODES,), jnp.float32),
                  jax.ShapeDtypeStruct((N_NODES,), jnp.float32)),
        mesh=mesh,
        compiler_params=pltpu.CompilerParams(needs_layout_passes=False,
                                             use_tc_tiling_on_sc=False),
        scratch_types=[
            pltpu.VMEM((3, CHUNK_CELLS), jnp.int32),
            pltpu.VMEM((3, CHUNK_CELLS), jnp.int32),
            pltpu.VMEM((3 * CHUNK_CELLS, 16), jnp.float32),
            pltpu.VMEM((2, CHUNK_CELLS), jnp.float32),
            pltpu.VMEM((6, CHUNK_CELLS), jnp.float32),
            pltpu.VMEM((3, CHUNK_CELLS), jnp.float32),
            pltpu.VMEM((3, CHUNK_CELLS), jnp.float32),
            pltpu.VMEM((3, CHUNK_CELLS), jnp.float32),
            pltpu.VMEM((32, 16), jnp.float32),
            pltpu.SemaphoreType.DMA,
            pltpu.SemaphoreType.DMA,
            pltpu.SemaphoreType.DMA,
            pltpu.VMEM_SHARED((N_NODES,), jnp.float32),
        ],
    )
    return kern(q, tri_t, cc_t, vp0, vp1, vp2, ffd_t, wtab, zeros)


def _combine_body(p0_ref, p1_ref, m_ref, o_ref):
    o_ref[...] = (p0_ref[...] + p1_ref[...]) * m_ref[...]


def _combine(p0, p1, inv_mass2d):
    return pl.pallas_call(
        _combine_body,
        out_shape=jax.ShapeDtypeStruct((1, N_NODES), jnp.float32),
    )(p0[None, :], p1[None, :], inv_mass2d)


@jax.jit
def kernel(u, t, triangulation, cell_centers, cell_local_vertex_pos,
           free_form_data, inv_mass, W, b):
    u2d = u[0]

    wv = W[9:].reshape(3, D, 3)
    wcat = jnp.transpose(wv, (1, 0, 2)).reshape(D, 9)
    wcat = jnp.pad(wcat, ((0, 0), (0, 7)))
    q = _compute_q(u2d, wcat)

    const = t[0, 0] * W[0] + b
    wrows = jnp.concatenate([W[1:9].reshape(-1), const,
                             jnp.zeros((5,), jnp.float32)])
    wtab = jnp.broadcast_to(wrows[:, None], (32, 16))

    zeros = jnp.zeros((N_NODES,), jnp.float32)

    tri_t = triangulation.T
    cc_t = cell_centers.T
    vp0 = cell_local_vertex_pos[:, 0, :].T
    vp1 = cell_local_vertex_pos[:, 1, :].T
    vp2 = cell_local_vertex_pos[:, 2, :].T
    ffd_t = free_form_data.T

    p0, p1 = _sc_scatter(q, tri_t, cc_t, vp0, vp1, vp2, ffd_t, wtab, zeros)

    return _combine(p0, p1, inv_mass[None, :])
